# baseline (speedup 1.0000x reference)
import functools

import jax
import jax.numpy as jnp
from jax.experimental import pallas as pl
from jax.experimental.pallas import tpu as pltpu

_SUPPORT = 0.05
_NB = 4
_MB = 4
_SPACING = 2.0 / (_NB - 1)
_INV_PI = 1.0 / 3.14159265358979323846


def _hat(c, center):
    return jnp.maximum(0.0, 1.0 - jnp.abs(c - center) * (1.0 / _SPACING))


def _atan2(y, x):
    ax = jnp.abs(x)
    ay = jnp.abs(y)
    hi = jnp.maximum(ax, ay)
    lo = jnp.minimum(ax, ay)
    z = lo / hi
    t = z * z
    p = z * (0.99997726 + t * (-0.33262347 + t * (0.19354346 + t * (
        -0.11643287 + t * (0.05265332 + t * -0.01172120)))))
    p = jnp.where(ay > ax, 1.57079632679 - p, p)
    p = jnp.where(x < 0.0, 3.14159265359 - p, p)
    p = jnp.where(hi == 0.0, 0.0, p)
    yneg = jax.lax.bitcast_convert_type(y, jnp.int32) < 0
    return jnp.where(yneg, -p, p)


def _mm_body(x_ref, w_ref, out_ref, *, relu):
    r = jnp.dot(x_ref[...], w_ref[...], preferred_element_type=jnp.float32)
    if relu:
        r = jnp.maximum(r, 0.0)
    out_ref[...] = r


def _mm(x, w, relu=False):
    return pl.pallas_call(
        functools.partial(_mm_body, relu=relu),
        out_shape=jax.ShapeDtypeStruct((x.shape[0], w.shape[1]), jnp.float32),
    )(x, w)


def _conv_body(ptT_ref, ps_ref, hsT_ref, *rest, sign, self_mask, relu,
               has_add, Tb, Sb, oc):
    if has_add:
        add_ref, out_ref = rest
    else:
        (out_ref,) = rest
    si = pl.program_id(1)
    ti = pl.program_id(0)
    ns = pl.num_programs(1)

    @pl.when(si == 0)
    def _init():
        out_ref[...] = jnp.zeros_like(out_ref)

    ptT = ptT_ref[...]
    ps = ps_ref[...]

    ty = ptT[1:2, :]
    sy = ps[:, 1:2]
    overlap = ((jnp.min(sy) <= jnp.max(ty) + _SUPPORT)
               & (jnp.min(ty) - _SUPPORT <= jnp.max(sy)))

    @pl.when(overlap)
    def _work():
        dx = ps[:, 0:1] - ptT[0:1, :]
        dy = sy - ty
        d2 = dx * dx + dy * dy
        mask = d2 <= _SUPPORT * _SUPPORT
        if self_mask:
            gs = si * Sb + jax.lax.broadcasted_iota(jnp.int32, (Sb, Tb), 0)
            gt = ti * Tb + jax.lax.broadcasted_iota(jnp.int32, (Sb, Tb), 1)
            mask = mask & (gt != gs)
        evx = jnp.clip(sign * dx * (1.0 / _SUPPORT), -1.0, 1.0)
        evy = jnp.clip(sign * dy * (1.0 / _SUPPORT), -1.0, 1.0)
        r = jnp.clip(jnp.sqrt(evx * evx + evy * evy + 1e-12), 0.0, 1.0)
        c1 = 2.0 * r - 1.0
        c2 = _atan2(evy, evx) * _INV_PI
        fm = mask.astype(jnp.float32)

        acc = jnp.zeros((oc, Tb), jnp.float32)
        bys = [_hat(c2, -1.0 + _SPACING * m).astype(jnp.bfloat16)
               for m in range(_MB)]
        for n in range(_NB):
            bxn = (_hat(c1, -1.0 + _SPACING * n) * fm).astype(jnp.bfloat16)
            for m in range(_MB):
                a = bxn * bys[m]
                acc = acc + jnp.dot(hsT_ref[n * _MB + m], a,
                                    preferred_element_type=jnp.float32)
        out_ref[...] += acc

    @pl.when(si == ns - 1)
    def _fin():
        res = out_ref[...]
        if has_add:
            res = res + add_ref[...]
        if relu:
            res = jnp.maximum(res, 0.0)
        out_ref[...] = res


def _conv(ptT, ps, hsT, add=None, *, sign, self_mask, relu, Tb=256, Sb=256):
    T = ptT.shape[1]
    S = ps.shape[0]
    oc = hsT.shape[1]
    grid = (T // Tb, S // Sb)
    in_specs = [
        pl.BlockSpec((2, Tb), lambda t, s: (0, t)),
        pl.BlockSpec((Sb, 2), lambda t, s: (s, 0)),
        pl.BlockSpec((_NB * _MB, oc, Sb), lambda t, s: (0, 0, s)),
    ]
    args = [ptT, ps, hsT]
    if add is not None:
        in_specs.append(pl.BlockSpec((oc, Tb), lambda t, s: (0, t)))
        args.append(add)
    body = functools.partial(
        _conv_body, sign=sign, self_mask=self_mask, relu=relu,
        has_add=add is not None, Tb=Tb, Sb=Sb, oc=oc)
    return pl.pallas_call(
        body,
        grid=grid,
        in_specs=in_specs,
        out_specs=pl.BlockSpec((oc, Tb), lambda t, s: (0, t)),
        out_shape=jax.ShapeDtypeStruct((oc, T), jnp.float32),
        compiler_params=pltpu.CompilerParams(
            dimension_semantics=("parallel", "arbitrary")),
    )(*args)


def _wrT(W, oc_pad=None):
    n, m, ci, co = W.shape
    if oc_pad is not None and oc_pad != co:
        W = jnp.pad(W, ((0, 0), (0, 0), (0, 0), (0, oc_pad - co)))
        co = oc_pad
    return jnp.transpose(W, (0, 1, 3, 2)).reshape(n * m * co, ci)


def _hsT(hsT2, oc):
    return hsT2.reshape(_NB * _MB, oc, hsT2.shape[1]).astype(jnp.bfloat16)


def kernel(fluidPositions, boundaryPositions, fluidFeatures, boundaryFeatures,
           W0, W1, W2, W3, Wfc0, Wfc1, Wfc2):
    forder = jnp.argsort(fluidPositions[:, 1])
    border = jnp.argsort(boundaryPositions[:, 1])
    fluidPositions = fluidPositions[forder]
    fluidFeatures = fluidFeatures[forder]
    boundaryPositions = boundaryPositions[border]
    boundaryFeatures = boundaryFeatures[border]

    pfT = jnp.transpose(fluidPositions)
    fT = jnp.transpose(fluidFeatures)
    bT = jnp.transpose(boundaryFeatures)

    linT = _mm(jnp.transpose(Wfc0), fT, relu=True)

    hsT1 = _hsT(_mm(_wrT(W1), bT), 32)
    bconvT = _conv(pfT, boundaryPositions, hsT1,
                   sign=1.0, self_mask=False, relu=True)

    hsT0 = _hsT(_mm(_wrT(W0), fT), 32)
    fconvT = _conv(pfT, fluidPositions, hsT0,
                   sign=-1.0, self_mask=True, relu=True)

    ansT = jnp.concatenate([linT, fconvT, bconvT], axis=0)

    fc1T = _mm(jnp.transpose(Wfc1), ansT)
    hsT2 = _hsT(_mm(_wrT(W2), ansT), 32)
    hiddenT = _conv(pfT, fluidPositions, hsT2, add=fc1T,
                    sign=-1.0, self_mask=True, relu=True)

    fc2T = _mm(jnp.transpose(jnp.pad(Wfc2, ((0, 0), (0, 6)))), hiddenT)
    hsT3 = _hsT(_mm(_wrT(W3, oc_pad=8), hiddenT), 8)
    outT = _conv(pfT, fluidPositions, hsT3, add=fc2T,
                 sign=-1.0, self_mask=True, relu=False)
    return jnp.transpose(outT[:2])[jnp.argsort(forder)]


# speedup vs baseline: 1.7073x; 1.7073x over previous
import functools

import jax
import jax.numpy as jnp
from jax.experimental import pallas as pl
from jax.experimental.pallas import tpu as pltpu

_SUPPORT = 0.05
_NB = 4
_MB = 4
_SPACING = 2.0 / (_NB - 1)
_INV_PI = 1.0 / 3.14159265358979323846


def _hat(c, center):
    return jnp.maximum(0.0, 1.0 - jnp.abs(c - center) * (1.0 / _SPACING))


def _atan2(y, x):
    ax = jnp.abs(x)
    ay = jnp.abs(y)
    hi = jnp.maximum(ax, ay)
    lo = jnp.minimum(ax, ay)
    z = lo / hi
    t = z * z
    p = z * (0.99997726 + t * (-0.33262347 + t * (0.19354346 + t * (
        -0.11643287 + t * (0.05265332 + t * -0.01172120)))))
    p = jnp.where(ay > ax, 1.57079632679 - p, p)
    p = jnp.where(x < 0.0, 3.14159265359 - p, p)
    p = jnp.where(hi == 0.0, 0.0, p)
    yneg = jax.lax.bitcast_convert_type(y, jnp.int32) < 0
    return jnp.where(yneg, -p, p)


def _mm_body(x_ref, w_ref, out_ref, *, relu):
    r = jnp.dot(x_ref[...], w_ref[...], preferred_element_type=jnp.float32)
    if relu:
        r = jnp.maximum(r, 0.0)
    out_ref[...] = r


def _mm(x, w, relu=False):
    return pl.pallas_call(
        functools.partial(_mm_body, relu=relu),
        out_shape=jax.ShapeDtypeStruct((x.shape[0], w.shape[1]), jnp.float32),
    )(x, w)


def _conv_body(ptT_ref, ps_ref, hsT_ref, *rest, sign, self_mask, relu,
               has_add, Tb, Sb, oc, nsc):
    if has_add:
        add_ref, out_ref = rest
    else:
        (out_ref,) = rest
    ti = pl.program_id(0)

    ptT = ptT_ref[...]
    ty = ptT[1:2, :]
    tymin = jnp.min(ty)
    tymax = jnp.max(ty)

    def chunk(c, acc):
        symin = jnp.min(ps_ref[c, 0:1, 1:2])
        symax = jnp.min(ps_ref[c, Sb - 1:Sb, 1:2])
        ov = (symin <= tymax + _SUPPORT) & (tymin - _SUPPORT <= symax)

        def work():
            ps = ps_ref[c]
            dx = ps[:, 0:1] - ptT[0:1, :]
            dy = ps[:, 1:2] - ty
            d2 = dx * dx + dy * dy
            mask = d2 <= _SUPPORT * _SUPPORT
            if self_mask:
                gs = c * Sb + jax.lax.broadcasted_iota(jnp.int32, (Sb, Tb), 0)
                gt = ti * Tb + jax.lax.broadcasted_iota(jnp.int32, (Sb, Tb), 1)
                mask = mask & (gt != gs)
            evx = jnp.clip(sign * dx * (1.0 / _SUPPORT), -1.0, 1.0)
            evy = jnp.clip(sign * dy * (1.0 / _SUPPORT), -1.0, 1.0)
            r = jnp.clip(jnp.sqrt(evx * evx + evy * evy + 1e-12), 0.0, 1.0)
            c1 = 2.0 * r - 1.0
            c2 = _atan2(evy, evx) * _INV_PI
            fm = mask.astype(jnp.float32)

            inner = acc
            bys = [_hat(c2, -1.0 + _SPACING * m).astype(jnp.bfloat16)
                   for m in range(_MB)]
            for n in range(_NB):
                bxn = (_hat(c1, -1.0 + _SPACING * n) * fm).astype(jnp.bfloat16)
                for m in range(_MB):
                    a = bxn * bys[m]
                    inner = inner + jnp.dot(hsT_ref[c, n * _MB + m], a,
                                            preferred_element_type=jnp.float32)
            return inner

        return jax.lax.cond(ov, work, lambda: acc)

    res = jax.lax.fori_loop(0, nsc, chunk,
                            jnp.zeros((oc, Tb), jnp.float32))
    if has_add:
        res = res + add_ref[...]
    if relu:
        res = jnp.maximum(res, 0.0)
    out_ref[...] = res


def _conv(ptT, ps3, hsT, add=None, *, sign, self_mask, relu, Tb=256, Sb=512):
    T = ptT.shape[1]
    nsc = ps3.shape[0]
    oc = hsT.shape[2]
    grid = (T // Tb,)
    in_specs = [
        pl.BlockSpec((2, Tb), lambda t: (0, t)),
        pl.BlockSpec(ps3.shape, lambda t: (0, 0, 0)),
        pl.BlockSpec(hsT.shape, lambda t: (0, 0, 0, 0)),
    ]
    args = [ptT, ps3, hsT]
    if add is not None:
        in_specs.append(pl.BlockSpec((oc, Tb), lambda t: (0, t)))
        args.append(add)
    body = functools.partial(
        _conv_body, sign=sign, self_mask=self_mask, relu=relu,
        has_add=add is not None, Tb=Tb, Sb=Sb, oc=oc, nsc=nsc)
    return pl.pallas_call(
        body,
        grid=grid,
        in_specs=in_specs,
        out_specs=pl.BlockSpec((oc, Tb), lambda t: (0, t)),
        out_shape=jax.ShapeDtypeStruct((oc, T), jnp.float32),
        compiler_params=pltpu.CompilerParams(
            dimension_semantics=("parallel",)),
    )(*args)


def _wrT(W, oc_pad=None):
    n, m, ci, co = W.shape
    if oc_pad is not None and oc_pad != co:
        W = jnp.pad(W, ((0, 0), (0, 0), (0, 0), (0, oc_pad - co)))
        co = oc_pad
    return jnp.transpose(W, (0, 1, 3, 2)).reshape(n * m * co, ci)


def _hsT(hsT2, oc, Sb=512):
    S = hsT2.shape[1]
    h = hsT2.reshape(_NB * _MB, oc, S // Sb, Sb)
    return jnp.transpose(h, (2, 0, 1, 3)).astype(jnp.bfloat16)


def _ps3(p, Sb=512):
    return p.reshape(p.shape[0] // Sb, Sb, 2)


def kernel(fluidPositions, boundaryPositions, fluidFeatures, boundaryFeatures,
           W0, W1, W2, W3, Wfc0, Wfc1, Wfc2):
    forder = jnp.argsort(fluidPositions[:, 1])
    border = jnp.argsort(boundaryPositions[:, 1])
    fluidPositions = fluidPositions[forder]
    fluidFeatures = fluidFeatures[forder]
    boundaryPositions = boundaryPositions[border]
    boundaryFeatures = boundaryFeatures[border]

    pfT = jnp.transpose(fluidPositions)
    fT = jnp.transpose(fluidFeatures)
    bT = jnp.transpose(boundaryFeatures)

    linT = _mm(jnp.transpose(Wfc0), fT, relu=True)

    hsT1 = _hsT(_mm(_wrT(W1), bT), 32)
    bconvT = _conv(pfT, _ps3(boundaryPositions), hsT1,
                   sign=1.0, self_mask=False, relu=True)

    hsT0 = _hsT(_mm(_wrT(W0), fT), 32)
    fconvT = _conv(pfT, _ps3(fluidPositions), hsT0,
                   sign=-1.0, self_mask=True, relu=True)

    ansT = jnp.concatenate([linT, fconvT, bconvT], axis=0)

    fc1T = _mm(jnp.transpose(Wfc1), ansT)
    hsT2 = _hsT(_mm(_wrT(W2), ansT), 32)
    hiddenT = _conv(pfT, _ps3(fluidPositions), hsT2, add=fc1T,
                    sign=-1.0, self_mask=True, relu=True)

    fc2T = _mm(jnp.transpose(jnp.pad(Wfc2, ((0, 0), (0, 6)))), hiddenT)
    hsT3 = _hsT(_mm(_wrT(W3, oc_pad=8), hiddenT), 8)
    outT = _conv(pfT, _ps3(fluidPositions), hsT3, add=fc2T,
                 sign=-1.0, self_mask=True, relu=False)
    return jnp.transpose(outT[:2])[jnp.argsort(forder)]


# speedup vs baseline: 1.7395x; 1.0188x over previous
import functools

import jax
import jax.numpy as jnp
from jax.experimental import pallas as pl
from jax.experimental.pallas import tpu as pltpu

_SUPPORT = 0.05
_NB = 4
_MB = 4
_SPACING = 2.0 / (_NB - 1)
_INV_PI = 1.0 / 3.14159265358979323846


def _hat(c, center):
    return jnp.maximum(0.0, 1.0 - jnp.abs(c - center) * (1.0 / _SPACING))


def _atan2(y, x):
    ax = jnp.abs(x)
    ay = jnp.abs(y)
    hi = jnp.maximum(ax, ay)
    lo = jnp.minimum(ax, ay)
    z = lo / hi
    t = z * z
    p = z * (0.99997726 + t * (-0.33262347 + t * (0.19354346 + t * (
        -0.11643287 + t * (0.05265332 + t * -0.01172120)))))
    p = jnp.where(ay > ax, 1.57079632679 - p, p)
    p = jnp.where(x < 0.0, 3.14159265359 - p, p)
    p = jnp.where(hi == 0.0, 0.0, p)
    yneg = jax.lax.bitcast_convert_type(y, jnp.int32) < 0
    return jnp.where(yneg, -p, p)


def _mm_body(x_ref, w_ref, out_ref, *, relu):
    r = jnp.dot(x_ref[...], w_ref[...], preferred_element_type=jnp.float32)
    if relu:
        r = jnp.maximum(r, 0.0)
    out_ref[...] = r


def _mm(x, w, relu=False):
    return pl.pallas_call(
        functools.partial(_mm_body, relu=relu),
        out_shape=jax.ShapeDtypeStruct((x.shape[0], w.shape[1]), jnp.float32),
    )(x, w)


def _conv_body(ptT_ref, ps_ref, hsT_ref, *rest, sign, self_mask, relu,
               has_add, Tb, Sb, oc, nsc):
    if has_add:
        add_ref, out_ref = rest
    else:
        (out_ref,) = rest
    ti = pl.program_id(0)

    ptT = ptT_ref[...]
    ty = ptT[1:2, :]
    tymin = jnp.min(ty)
    tymax = jnp.max(ty)

    def chunk(c, acc):
        symin = jnp.min(ps_ref[c, 0:1, 1:2])
        symax = jnp.min(ps_ref[c, Sb - 1:Sb, 1:2])
        ov = (symin <= tymax + _SUPPORT) & (tymin - _SUPPORT <= symax)

        def work():
            ps = ps_ref[c]
            dx = ps[:, 0:1] - ptT[0:1, :]
            dy = ps[:, 1:2] - ty
            d2 = dx * dx + dy * dy
            mask = d2 <= _SUPPORT * _SUPPORT
            if self_mask:
                gs = c * Sb + jax.lax.broadcasted_iota(jnp.int32, (Sb, Tb), 0)
                gt = ti * Tb + jax.lax.broadcasted_iota(jnp.int32, (Sb, Tb), 1)
                mask = mask & (gt != gs)
            evx = jnp.clip(sign * dx * (1.0 / _SUPPORT), -1.0, 1.0)
            evy = jnp.clip(sign * dy * (1.0 / _SUPPORT), -1.0, 1.0)
            r = jnp.clip(jnp.sqrt(evx * evx + evy * evy + 1e-12), 0.0, 1.0)
            c1 = 2.0 * r - 1.0
            c2 = _atan2(evy, evx) * _INV_PI
            fm = mask.astype(jnp.float32)

            inner = acc
            bys = [_hat(c2, -1.0 + _SPACING * m).astype(jnp.bfloat16)
                   for m in range(_MB)]
            for n in range(_NB):
                bxn = (_hat(c1, -1.0 + _SPACING * n) * fm).astype(jnp.bfloat16)
                for m in range(_MB):
                    a = bxn * bys[m]
                    inner = inner + jnp.dot(hsT_ref[c, n * _MB + m], a,
                                            preferred_element_type=jnp.float32)
            return inner

        return jax.lax.cond(ov, work, lambda: acc)

    res = jax.lax.fori_loop(0, nsc, chunk,
                            jnp.zeros((oc, Tb), jnp.float32))
    if has_add:
        res = res + add_ref[...]
    if relu:
        res = jnp.maximum(res, 0.0)
    out_ref[...] = res


def _conv(ptT, ps3, hsT, add=None, *, sign, self_mask, relu, Tb=256, Sb=256):
    T = ptT.shape[1]
    nsc = ps3.shape[0]
    oc = hsT.shape[2]
    grid = (T // Tb,)
    in_specs = [
        pl.BlockSpec((2, Tb), lambda t: (0, t)),
        pl.BlockSpec(ps3.shape, lambda t: (0, 0, 0)),
        pl.BlockSpec(hsT.shape, lambda t: (0, 0, 0, 0)),
    ]
    args = [ptT, ps3, hsT]
    if add is not None:
        in_specs.append(pl.BlockSpec((oc, Tb), lambda t: (0, t)))
        args.append(add)
    body = functools.partial(
        _conv_body, sign=sign, self_mask=self_mask, relu=relu,
        has_add=add is not None, Tb=Tb, Sb=Sb, oc=oc, nsc=nsc)
    return pl.pallas_call(
        body,
        grid=grid,
        in_specs=in_specs,
        out_specs=pl.BlockSpec((oc, Tb), lambda t: (0, t)),
        out_shape=jax.ShapeDtypeStruct((oc, T), jnp.float32),
        compiler_params=pltpu.CompilerParams(
            dimension_semantics=("parallel",)),
    )(*args)


def _wrT(W, oc_pad=None):
    n, m, ci, co = W.shape
    if oc_pad is not None and oc_pad != co:
        W = jnp.pad(W, ((0, 0), (0, 0), (0, 0), (0, oc_pad - co)))
        co = oc_pad
    return jnp.transpose(W, (0, 1, 3, 2)).reshape(n * m * co, ci)


def _hsT(hsT2, oc, Sb=256):
    S = hsT2.shape[1]
    h = hsT2.reshape(_NB * _MB, oc, S // Sb, Sb)
    return jnp.transpose(h, (2, 0, 1, 3)).astype(jnp.bfloat16)


def _ps3(p, Sb=256):
    return p.reshape(p.shape[0] // Sb, Sb, 2)


def kernel(fluidPositions, boundaryPositions, fluidFeatures, boundaryFeatures,
           W0, W1, W2, W3, Wfc0, Wfc1, Wfc2):
    forder = jnp.argsort(fluidPositions[:, 1])
    border = jnp.argsort(boundaryPositions[:, 1])
    fluidPositions = fluidPositions[forder]
    fluidFeatures = fluidFeatures[forder]
    boundaryPositions = boundaryPositions[border]
    boundaryFeatures = boundaryFeatures[border]

    pfT = jnp.transpose(fluidPositions)
    fT = jnp.transpose(fluidFeatures)
    bT = jnp.transpose(boundaryFeatures)

    linT = _mm(jnp.transpose(Wfc0), fT, relu=True)

    hsT1 = _hsT(_mm(_wrT(W1), bT), 32)
    bconvT = _conv(pfT, _ps3(boundaryPositions), hsT1,
                   sign=1.0, self_mask=False, relu=True)

    hsT0 = _hsT(_mm(_wrT(W0), fT), 32)
    fconvT = _conv(pfT, _ps3(fluidPositions), hsT0,
                   sign=-1.0, self_mask=True, relu=True)

    ansT = jnp.concatenate([linT, fconvT, bconvT], axis=0)

    fc1T = _mm(jnp.transpose(Wfc1), ansT)
    hsT2 = _hsT(_mm(_wrT(W2), ansT), 32)
    hiddenT = _conv(pfT, _ps3(fluidPositions), hsT2, add=fc1T,
                    sign=-1.0, self_mask=True, relu=True)

    fc2T = _mm(jnp.transpose(jnp.pad(Wfc2, ((0, 0), (0, 6)))), hiddenT)
    hsT3 = _hsT(_mm(_wrT(W3, oc_pad=8), hiddenT), 8)
    outT = _conv(pfT, _ps3(fluidPositions), hsT3, add=fc2T,
                 sign=-1.0, self_mask=True, relu=False)
    return jnp.transpose(outT[:2])[jnp.argsort(forder)]
